# baseline (speedup 1.0000x reference)
import functools

import jax
import jax.numpy as jnp
from jax import lax
from jax.experimental import pallas as pl
from jax.experimental.pallas import tpu as pltpu
from jax.experimental.pallas import tpu_sc as plsc

N = 10000
NP = 10240
E = 320000
F = 64
FP = 2 * F
ALPHA = 0.2

NC = 2
NS = 16
NW = NC * NS
EPW = E // NW
SUB2 = 80
NSUB2 = EPW // SUB2


def _tc_head_fn(H, P, NB, x_r, w_r, asrc_r, atgt_r, h_r, ss_r, st_r, m_r,
                mx_s):
    i = pl.program_id(0)
    for hh in range(2 * P):
        p, half = divmod(hh, 2)
        if hh < H:
            hb = jnp.dot(x_r[...], w_r[hh], preferred_element_type=jnp.float32)
            h_r[p, :, F * half:F * (half + 1)] = hb
            s1 = jnp.dot(hb, asrc_r[hh], preferred_element_type=jnp.float32)
            s2 = jnp.dot(hb, atgt_r[hh], preferred_element_type=jnp.float32)
            b1 = jnp.max(s1)
            b2 = jnp.max(s2)
        else:
            h_r[p, :, F * half:F * (half + 1)] = jnp.zeros(
                (x_r.shape[0], F), jnp.float32)
            s1 = jnp.zeros((x_r.shape[0],), jnp.float32)
            s2 = s1
            b1 = jnp.float32(0.0)
            b2 = jnp.float32(0.0)
        ss_r[hh] = s1
        st_r[hh] = s2

        @pl.when(i == 0)
        def _():
            mx_s[hh, 0] = b1
            mx_s[hh, 1] = b2

        @pl.when(i > 0)
        def _():
            mx_s[hh, 0] = jnp.maximum(mx_s[hh, 0], b1)
            mx_s[hh, 1] = jnp.maximum(mx_s[hh, 1], b2)

    @pl.when(i == NB - 1)
    def _():
        for hh in range(2 * P):
            m_r[hh] = jnp.full((128,),
                               jnp.maximum(mx_s[hh, 0] + mx_s[hh, 1], 0.0),
                               dtype=jnp.float32)


def _tc_head(H, K, BN=2048):
    P = (H + 1) // 2
    NB = NP // BN
    return pl.pallas_call(
        functools.partial(_tc_head_fn, H, P, NB),
        grid=(NB,),
        in_specs=[
            pl.BlockSpec((BN, K), lambda i: (i, 0)),
            pl.BlockSpec((H, K, F), lambda i: (0, 0, 0)),
            pl.BlockSpec((H, F), lambda i: (0, 0)),
            pl.BlockSpec((H, F), lambda i: (0, 0)),
        ],
        out_specs=[
            pl.BlockSpec((P, BN, FP), lambda i: (0, i, 0)),
            pl.BlockSpec((2 * P, BN), lambda i: (0, i)),
            pl.BlockSpec((2 * P, BN), lambda i: (0, i)),
            pl.BlockSpec((2 * P, 128), lambda i: (0, 0)),
        ],
        out_shape=[
            jax.ShapeDtypeStruct((P, NP, FP), jnp.float32),
            jax.ShapeDtypeStruct((2 * P, NP), jnp.float32),
            jax.ShapeDtypeStruct((2 * P, NP), jnp.float32),
            jax.ShapeDtypeStruct((2 * P, 128), jnp.float32),
        ],
        scratch_shapes=[pltpu.SMEM((2 * P, 2), jnp.float32)],
    )


def _sc_scores_body(P, ssrc_hbm, stgt_hbm, m_hbm, src_hbm, tgt_hbm,
                    w_out, es_out,
                    ssrc0_t, stgt0_t, ssrc1_t, stgt1_t, es0_t, es1_t,
                    src_w, tgt_w, w0_b, w1_b, m_v):
    cid = lax.axis_index("c")
    sid = lax.axis_index("s")
    wid = sid * NC + cid
    wbase = pl.multiple_of(wid * EPW, 8)

    pltpu.sync_copy(src_hbm.at[pl.ds(wbase, EPW)], src_w)
    pltpu.sync_copy(tgt_hbm.at[pl.ds(wbase, EPW)], tgt_w)

    for p in range(P):
        pltpu.sync_copy(ssrc_hbm.at[2 * p], ssrc0_t)
        pltpu.sync_copy(stgt_hbm.at[2 * p], stgt0_t)
        pltpu.sync_copy(ssrc_hbm.at[2 * p + 1], ssrc1_t)
        pltpu.sync_copy(stgt_hbm.at[2 * p + 1], stgt1_t)
        pltpu.sync_copy(m_hbm.at[2 * p], m_v)
        m0 = m_v[pl.ds(0, 16)]
        pltpu.sync_copy(m_hbm.at[2 * p + 1], m_v)
        m1 = m_v[pl.ds(0, 16)]

        def _zero_es(i, carry):
            es0_t[pl.ds(16 * i, 16)] = jnp.zeros((16,), jnp.float32)
            es1_t[pl.ds(16 * i, 16)] = jnp.zeros((16,), jnp.float32)
            return carry
        lax.fori_loop(0, NP // 16, _zero_es, 0)

        @plsc.parallel_loop(0, EPW // 16, 1, unroll=4)
        def _group(g):
            goff = pl.multiple_of(g * 16, 8)
            i_s = src_w[pl.ds(goff, 16)]
            i_t = tgt_w[pl.ds(goff, 16)]
            s1 = plsc.load_gather(ssrc0_t, [i_s])
            s2 = plsc.load_gather(stgt0_t, [i_t])
            e0 = s1 + s2
            e0 = jnp.where(e0 > 0, e0, ALPHA * e0)
            w0 = jnp.exp(e0 - m0)
            w0_b[pl.ds(goff, 16)] = w0
            plsc.addupdate_scatter(es0_t, [i_s], w0)
            s3 = plsc.load_gather(ssrc1_t, [i_s])
            s4 = plsc.load_gather(stgt1_t, [i_t])
            e1 = s3 + s4
            e1 = jnp.where(e1 > 0, e1, ALPHA * e1)
            w1 = jnp.exp(e1 - m1)
            w1_b[pl.ds(goff, 16)] = w1
            plsc.addupdate_scatter(es1_t, [i_s], w1)

        pltpu.sync_copy(w0_b, w_out.at[pl.ds((2 * p) * E + wbase, EPW)])
        pltpu.sync_copy(w1_b, w_out.at[pl.ds((2 * p + 1) * E + wbase, EPW)])
        pltpu.sync_copy(es0_t, es_out.at[(2 * p) * NW + wid])
        pltpu.sync_copy(es1_t, es_out.at[(2 * p + 1) * NW + wid])


def _sc_scores(P):
    mesh = plsc.VectorSubcoreMesh(core_axis_name="c", subcore_axis_name="s")
    return pl.kernel(
        functools.partial(_sc_scores_body, P),
        mesh=mesh,
        compiler_params=pltpu.CompilerParams(needs_layout_passes=False),
        out_type=[
            jax.ShapeDtypeStruct((2 * P * E,), jnp.float32),
            jax.ShapeDtypeStruct((2 * P * NW, NP), jnp.float32),
        ],
        scratch_types=[
            pltpu.VMEM((NP,), jnp.float32),
            pltpu.VMEM((NP,), jnp.float32),
            pltpu.VMEM((NP,), jnp.float32),
            pltpu.VMEM((NP,), jnp.float32),
            pltpu.VMEM((NP,), jnp.float32),
            pltpu.VMEM((NP,), jnp.float32),
            pltpu.VMEM((EPW,), jnp.int32),
            pltpu.VMEM((EPW,), jnp.int32),
            pltpu.VMEM((EPW,), jnp.float32),
            pltpu.VMEM((EPW,), jnp.float32),
            pltpu.VMEM((128,), jnp.float32),
        ],
    )


def _sc_rows_body(P, h2_hbm, src2d_hbm, tgt_hbm, w_hbm, acc_out,
                  tgt_adj, src2, w0a, w1a, w0b, w1b, rows_a, rows_b,
                  acc_sh, sga, sgb, swa0, swa1, swb0, swb1, ssa, ssb):
    cid = lax.axis_index("c")
    sid = lax.axis_index("s")
    wid = sid * NC + cid
    wbase = pl.multiple_of(wid * EPW, 8)

    pltpu.sync_copy(tgt_hbm.at[pl.ds(wbase, EPW)], tgt_adj)
    pltpu.sync_copy(src2d_hbm.at[wid], src2)

    stripe = NP // NS
    sbase = sid * stripe

    def _issue(c_dyn, p, rows, w0, w1, sg, sw0, sw1):
        coff = pl.multiple_of(c_dyn * SUB2, 8)
        g = pltpu.async_copy(h2_hbm.at[tgt_adj.at[pl.ds(coff, SUB2)]],
                             rows, sg)
        a0 = pltpu.async_copy(
            w_hbm.at[pl.ds((2 * p) * E + wbase + coff, SUB2)], w0, sw0)
        a1 = pltpu.async_copy(
            w_hbm.at[pl.ds((2 * p + 1) * E + wbase + coff, SUB2)], w1, sw1)
        return g, a0, a1

    def _scale(rbuf, w0_v, w1_v):
        for g in range(SUB2 // 16):
            w0g = w0_v[pl.ds(16 * g, 16)]
            w1g = w1_v[pl.ds(16 * g, 16)]
            for l in range(16):
                ei = 16 * g + l
                ws0 = w0g[l]
                ws1 = w1g[l]
                for j in range(FP // 16):
                    ws = ws0 if j < F // 16 else ws1
                    rbuf[ei, pl.ds(16 * j, 16)] = (
                        rbuf[ei, pl.ds(16 * j, 16)] * ws)

    def _wait(rows, w0, w1, sg, sw0, sw1):
        pltpu.make_async_copy(
            h2_hbm.at[tgt_adj.at[pl.ds(0, SUB2)]], rows, sg).wait()
        pltpu.make_async_copy(w_hbm.at[pl.ds(0, SUB2)], w0, sw0).wait()
        pltpu.make_async_copy(w_hbm.at[pl.ds(0, SUB2)], w1, sw1).wait()

    def _wait_scat(rows, ss):
        pltpu.make_async_copy(rows, acc_sh.at[src2.at[0]], ss).wait()

    for p in range(P):
        if p > 0:
            def _adj(i, carry):
                tgt_adj[pl.ds(16 * i, 16)] = tgt_adj[pl.ds(16 * i, 16)] + NP
                return carry
            lax.fori_loop(0, EPW // 16, _adj, 0)

        def _zero_rv(i, carry):
            for j in range(FP // 16):
                rows_b[i, pl.ds(16 * j, 16)] = jnp.zeros((16,), jnp.float32)
            return carry
        lax.fori_loop(0, SUB2, _zero_rv, 0)
        for zo in range(0, stripe, SUB2):
            pltpu.sync_copy(rows_b, acc_sh.at[pl.ds(sbase + zo, SUB2)])
        plsc.subcore_barrier()

        _issue(0, p, rows_a, w0a, w1a, sga, swa0, swa1)

        def _pair(i, carry):
            c0 = 2 * i
            _issue(c0 + 1, p, rows_b, w0b, w1b, sgb, swb0, swb1)
            _wait(rows_a, w0a, w1a, sga, swa0, swa1)
            _scale(rows_a, w0a, w1a)
            pltpu.sync_copy(rows_a, acc_sh.at[src2.at[c0]], add=True)
            _issue(c0 + 2, p, rows_a, w0a, w1a, sga, swa0, swa1)
            _wait(rows_b, w0b, w1b, sgb, swb0, swb1)
            _scale(rows_b, w0b, w1b)
            pltpu.sync_copy(rows_b, acc_sh.at[src2.at[c0 + 1]], add=True)
            return carry
        lax.fori_loop(0, NSUB2 // 2, _pair, 0)
        _wait(rows_a, w0a, w1a, sga, swa0, swa1)
        _scale(rows_a, w0a, w1a)
        pltpu.sync_copy(rows_a, acc_sh.at[src2.at[NSUB2 - 1]], add=True)

        plsc.subcore_barrier()

        pltpu.sync_copy(acc_sh.at[pl.ds(sbase, stripe)],
                        acc_out.at[pl.ds((p * NC + cid) * NP + sbase, stripe)])
        plsc.subcore_barrier()


def _sc_rows(P):
    mesh = plsc.VectorSubcoreMesh(core_axis_name="c", subcore_axis_name="s")
    return pl.kernel(
        functools.partial(_sc_rows_body, P),
        mesh=mesh,
        compiler_params=pltpu.CompilerParams(needs_layout_passes=False),
        out_type=jax.ShapeDtypeStruct((P * NC * NP, FP), jnp.float32),
        scratch_types=[
            pltpu.VMEM((EPW,), jnp.int32),
            pltpu.VMEM((NSUB2, SUB2), jnp.int32),
            pltpu.VMEM((SUB2,), jnp.float32),
            pltpu.VMEM((SUB2,), jnp.float32),
            pltpu.VMEM((SUB2,), jnp.float32),
            pltpu.VMEM((SUB2,), jnp.float32),
            pltpu.VMEM((SUB2, FP), jnp.float32),
            pltpu.VMEM((SUB2, FP), jnp.float32),
            pltpu.VMEM_SHARED((NP, FP), jnp.float32),
            pltpu.SemaphoreType.DMA,
            pltpu.SemaphoreType.DMA,
            pltpu.SemaphoreType.DMA,
            pltpu.SemaphoreType.DMA,
            pltpu.SemaphoreType.DMA,
            pltpu.SemaphoreType.DMA,
            pltpu.SemaphoreType.DMA,
            pltpu.SemaphoreType.DMA,
        ],
    )


def _tc_epi_fn(H, act, acc_r, es_r, b_r, out_r):
    for h in range(H):
        p, half = divmod(h, 2)
        num = (acc_r[2 * p, :, F * half:F * (half + 1)]
               + acc_r[2 * p + 1, :, F * half:F * (half + 1)])
        den = jnp.sum(es_r[pl.ds(NW * h, NW)], axis=0)
        den = den[:, None]
        hp = jnp.where(den > 0, num / den, 0.0) + b_r[h]
        if act == "elu":
            hp = jnp.where(hp > 0, hp, jnp.exp(hp) - 1.0)
            out_r[:, F * h:F * (h + 1)] = hp
        else:
            out_r[...] = jax.nn.sigmoid(hp)


def _tc_epi(H, act, BN=2048):
    P = (H + 1) // 2
    NB = NP // BN
    return pl.pallas_call(
        functools.partial(_tc_epi_fn, H, act),
        grid=(NB,),
        in_specs=[
            pl.BlockSpec((P * NC, BN, FP), lambda i: (0, i, 0)),
            pl.BlockSpec((2 * P * NW, BN), lambda i: (0, i)),
            pl.BlockSpec((H, F), lambda i: (0, 0)),
        ],
        out_specs=pl.BlockSpec((BN, F * (H if act == "elu" else 1)),
                               lambda i: (i, 0)),
        out_shape=jax.ShapeDtypeStruct((NP, F * (H if act == "elu" else 1)),
                                       jnp.float32),
    )


def _tc_fused_fn(H1, H2, P2, NB, acc_r, es_r, b_r, w_r, asrc_r, atgt_r,
                 h_r, ss_r, st_r, m_r, mx_s):
    i = pl.program_id(0)
    cols = []
    for h in range(H1):
        p, half = divmod(h, 2)
        num = (acc_r[2 * p, :, F * half:F * (half + 1)]
               + acc_r[2 * p + 1, :, F * half:F * (half + 1)])
        den = jnp.sum(es_r[pl.ds(NW * h, NW)], axis=0)[:, None]
        hp = jnp.where(den > 0, num / den, 0.0) + b_r[h]
        cols.append(jnp.where(hp > 0, hp, jnp.exp(hp) - 1.0))
    x = jnp.concatenate(cols, axis=1)

    for hh in range(2 * P2):
        p, half = divmod(hh, 2)
        if hh < H2:
            hb = jnp.dot(x, w_r[hh], preferred_element_type=jnp.float32)
            h_r[p, :, F * half:F * (half + 1)] = hb
            s1 = jnp.dot(hb, asrc_r[hh], preferred_element_type=jnp.float32)
            s2 = jnp.dot(hb, atgt_r[hh], preferred_element_type=jnp.float32)
            b1 = jnp.max(s1)
            b2 = jnp.max(s2)
        else:
            h_r[p, :, F * half:F * (half + 1)] = jnp.zeros(
                (x.shape[0], F), jnp.float32)
            s1 = jnp.zeros((x.shape[0],), jnp.float32)
            s2 = s1
            b1 = jnp.float32(0.0)
            b2 = jnp.float32(0.0)
        ss_r[hh] = s1
        st_r[hh] = s2

        @pl.when(i == 0)
        def _():
            mx_s[hh, 0] = b1
            mx_s[hh, 1] = b2

        @pl.when(i > 0)
        def _():
            mx_s[hh, 0] = jnp.maximum(mx_s[hh, 0], b1)
            mx_s[hh, 1] = jnp.maximum(mx_s[hh, 1], b2)

    @pl.when(i == NB - 1)
    def _():
        for hh in range(2 * P2):
            m_r[hh] = jnp.full((128,),
                               jnp.maximum(mx_s[hh, 0] + mx_s[hh, 1], 0.0),
                               dtype=jnp.float32)


def _tc_fused(H1, H2, K2, BN=2048):
    P1 = (H1 + 1) // 2
    P2 = (H2 + 1) // 2
    NB = NP // BN
    return pl.pallas_call(
        functools.partial(_tc_fused_fn, H1, H2, P2, NB),
        grid=(NB,),
        in_specs=[
            pl.BlockSpec((P1 * NC, BN, FP), lambda i: (0, i, 0)),
            pl.BlockSpec((2 * P1 * NW, BN), lambda i: (0, i)),
            pl.BlockSpec((H1, F), lambda i: (0, 0)),
            pl.BlockSpec((H2, K2, F), lambda i: (0, 0, 0)),
            pl.BlockSpec((H2, F), lambda i: (0, 0)),
            pl.BlockSpec((H2, F), lambda i: (0, 0)),
        ],
        out_specs=[
            pl.BlockSpec((P2, BN, FP), lambda i: (0, i, 0)),
            pl.BlockSpec((2 * P2, BN), lambda i: (0, i)),
            pl.BlockSpec((2 * P2, BN), lambda i: (0, i)),
            pl.BlockSpec((2 * P2, 128), lambda i: (0, 0)),
        ],
        out_shape=[
            jax.ShapeDtypeStruct((P2, NP, FP), jnp.float32),
            jax.ShapeDtypeStruct((2 * P2, NP), jnp.float32),
            jax.ShapeDtypeStruct((2 * P2, NP), jnp.float32),
            jax.ShapeDtypeStruct((2 * P2, 128), jnp.float32),
        ],
        scratch_shapes=[pltpu.SMEM((2 * P2, 2), jnp.float32)],
    )


def _sc_stage(P, h3, ssrc, stgt, m, src, src2d, tgt):
    h2 = h3.reshape(P * NP, FP)
    w_e, es = _sc_scores(P)(ssrc, stgt, m, src, tgt)
    acc_flat = _sc_rows(P)(h2, src2d, tgt, w_e)
    return acc_flat.reshape(P * NC, NP, FP), es


def _split_a(a):
    return a[:, :F, 0], a[:, F:, 0]


def kernel(x, edge_index, W1, a1, b1, W2, a2, b2, W3, a3, b3):
    src = edge_index[0].astype(jnp.int32)
    tgt = edge_index[1].astype(jnp.int32)
    src2d = src.reshape(NW, NSUB2, SUB2)
    xp = jnp.pad(x, ((0, NP - N), (0, 0)))

    a1s, a1t = _split_a(a1)
    a2s, a2t = _split_a(a2)
    a3s, a3t = _split_a(a3)

    h3, ss, st, m = _tc_head(4, 128)(xp, W1, a1s, a1t)
    acc1, es1 = _sc_stage(2, h3, ss, st, m, src, src2d, tgt)

    h3, ss, st, m = _tc_fused(4, 4, 256)(acc1, es1, b1, W2, a2s, a2t)
    acc2, es2 = _sc_stage(2, h3, ss, st, m, src, src2d, tgt)

    h3, ss, st, m = _tc_fused(4, 1, 256)(acc2, es2, b2, W3, a3s, a3t)
    acc3, es3 = _sc_stage(1, h3, ss, st, m, src, src2d, tgt)

    out = _tc_epi(1, "sigmoid")(acc3, es3, b3)
    return out[:N]


# speedup vs baseline: 1.0202x; 1.0202x over previous
import functools

import jax
import jax.numpy as jnp
from jax import lax
from jax.experimental import pallas as pl
from jax.experimental.pallas import tpu as pltpu
from jax.experimental.pallas import tpu_sc as plsc

N = 10000
NP = 10240
E = 320000
F = 64
FP = 2 * F
ALPHA = 0.2

NC = 2
NS = 16
NW = NC * NS
EPW = E // NW
SUB2 = 80
NSUB2 = EPW // SUB2


def _tc_head_fn(H, P, NB, x_r, w_r, asrc_r, atgt_r, h_r, ss_r, st_r, m_r,
                mx_s):
    i = pl.program_id(0)
    for hh in range(2 * P):
        p, half = divmod(hh, 2)
        if hh < H:
            hb = jnp.dot(x_r[...], w_r[hh], preferred_element_type=jnp.float32)
            h_r[p, :, F * half:F * (half + 1)] = hb
            s1 = jnp.dot(hb, asrc_r[hh], preferred_element_type=jnp.float32)
            s2 = jnp.dot(hb, atgt_r[hh], preferred_element_type=jnp.float32)
            b1 = jnp.max(s1)
            b2 = jnp.max(s2)
        else:
            h_r[p, :, F * half:F * (half + 1)] = jnp.zeros(
                (x_r.shape[0], F), jnp.float32)
            s1 = jnp.zeros((x_r.shape[0],), jnp.float32)
            s2 = s1
            b1 = jnp.float32(0.0)
            b2 = jnp.float32(0.0)
        ss_r[hh] = s1
        st_r[hh] = s2

        @pl.when(i == 0)
        def _():
            mx_s[hh, 0] = b1
            mx_s[hh, 1] = b2

        @pl.when(i > 0)
        def _():
            mx_s[hh, 0] = jnp.maximum(mx_s[hh, 0], b1)
            mx_s[hh, 1] = jnp.maximum(mx_s[hh, 1], b2)

    @pl.when(i == NB - 1)
    def _():
        for hh in range(2 * P):
            m_r[hh] = jnp.full((128,),
                               jnp.maximum(mx_s[hh, 0] + mx_s[hh, 1], 0.0),
                               dtype=jnp.float32)


def _tc_head(H, K, BN=2048):
    P = (H + 1) // 2
    NB = NP // BN
    return pl.pallas_call(
        functools.partial(_tc_head_fn, H, P, NB),
        grid=(NB,),
        in_specs=[
            pl.BlockSpec((BN, K), lambda i: (i, 0)),
            pl.BlockSpec((H, K, F), lambda i: (0, 0, 0)),
            pl.BlockSpec((H, F), lambda i: (0, 0)),
            pl.BlockSpec((H, F), lambda i: (0, 0)),
        ],
        out_specs=[
            pl.BlockSpec((P, BN, FP), lambda i: (0, i, 0)),
            pl.BlockSpec((2 * P, BN), lambda i: (0, i)),
            pl.BlockSpec((2 * P, BN), lambda i: (0, i)),
            pl.BlockSpec((2 * P, 128), lambda i: (0, 0)),
        ],
        out_shape=[
            jax.ShapeDtypeStruct((P, NP, FP), jnp.float32),
            jax.ShapeDtypeStruct((2 * P, NP), jnp.float32),
            jax.ShapeDtypeStruct((2 * P, NP), jnp.float32),
            jax.ShapeDtypeStruct((2 * P, 128), jnp.float32),
        ],
        scratch_shapes=[pltpu.SMEM((2 * P, 2), jnp.float32)],
    )


def _sc_scores_body(P, ssrc_hbm, stgt_hbm, m_hbm, src_hbm, tgt_hbm,
                    w_out, es_out,
                    ssrc0_t, stgt0_t, ssrc1_t, stgt1_t, es0_t, es1_t,
                    src_w, tgt_w, w0_b, w1_b, m_v):
    cid = lax.axis_index("c")
    sid = lax.axis_index("s")
    wid = sid * NC + cid
    wbase = pl.multiple_of(wid * EPW, 8)

    pltpu.sync_copy(src_hbm.at[pl.ds(wbase, EPW)], src_w)
    pltpu.sync_copy(tgt_hbm.at[pl.ds(wbase, EPW)], tgt_w)

    for p in range(P):
        pltpu.sync_copy(ssrc_hbm.at[2 * p], ssrc0_t)
        pltpu.sync_copy(stgt_hbm.at[2 * p], stgt0_t)
        pltpu.sync_copy(ssrc_hbm.at[2 * p + 1], ssrc1_t)
        pltpu.sync_copy(stgt_hbm.at[2 * p + 1], stgt1_t)
        pltpu.sync_copy(m_hbm.at[2 * p], m_v)
        m0 = m_v[pl.ds(0, 16)]
        pltpu.sync_copy(m_hbm.at[2 * p + 1], m_v)
        m1 = m_v[pl.ds(0, 16)]

        @plsc.parallel_loop(0, NP // 16, 1, unroll=4)
        def _zero_es(i):
            es0_t[pl.ds(16 * i, 16)] = jnp.zeros((16,), jnp.float32)
            es1_t[pl.ds(16 * i, 16)] = jnp.zeros((16,), jnp.float32)

        @plsc.parallel_loop(0, EPW // 16, 1, unroll=2)
        def _group(g):
            goff = pl.multiple_of(g * 16, 8)
            i_s = src_w[pl.ds(goff, 16)]
            i_t = tgt_w[pl.ds(goff, 16)]
            s1 = plsc.load_gather(ssrc0_t, [i_s])
            s2 = plsc.load_gather(stgt0_t, [i_t])
            e0 = s1 + s2
            e0 = jnp.where(e0 > 0, e0, ALPHA * e0)
            w0 = jnp.exp(e0 - m0)
            w0_b[pl.ds(goff, 16)] = w0
            plsc.addupdate_scatter(es0_t, [i_s], w0)
            s3 = plsc.load_gather(ssrc1_t, [i_s])
            s4 = plsc.load_gather(stgt1_t, [i_t])
            e1 = s3 + s4
            e1 = jnp.where(e1 > 0, e1, ALPHA * e1)
            w1 = jnp.exp(e1 - m1)
            w1_b[pl.ds(goff, 16)] = w1
            plsc.addupdate_scatter(es1_t, [i_s], w1)

        pltpu.sync_copy(w0_b, w_out.at[pl.ds((2 * p) * E + wbase, EPW)])
        pltpu.sync_copy(w1_b, w_out.at[pl.ds((2 * p + 1) * E + wbase, EPW)])
        pltpu.sync_copy(es0_t, es_out.at[(2 * p) * NW + wid])
        pltpu.sync_copy(es1_t, es_out.at[(2 * p + 1) * NW + wid])


def _sc_scores(P):
    mesh = plsc.VectorSubcoreMesh(core_axis_name="c", subcore_axis_name="s")
    return pl.kernel(
        functools.partial(_sc_scores_body, P),
        mesh=mesh,
        compiler_params=pltpu.CompilerParams(needs_layout_passes=False),
        out_type=[
            jax.ShapeDtypeStruct((2 * P * E,), jnp.float32),
            jax.ShapeDtypeStruct((2 * P * NW, NP), jnp.float32),
        ],
        scratch_types=[
            pltpu.VMEM((NP,), jnp.float32),
            pltpu.VMEM((NP,), jnp.float32),
            pltpu.VMEM((NP,), jnp.float32),
            pltpu.VMEM((NP,), jnp.float32),
            pltpu.VMEM((NP,), jnp.float32),
            pltpu.VMEM((NP,), jnp.float32),
            pltpu.VMEM((EPW,), jnp.int32),
            pltpu.VMEM((EPW,), jnp.int32),
            pltpu.VMEM((EPW,), jnp.float32),
            pltpu.VMEM((EPW,), jnp.float32),
            pltpu.VMEM((128,), jnp.float32),
        ],
    )


def _sc_rows_body(P, h2_hbm, src2d_hbm, tgt_hbm, w_hbm, acc_out,
                  tgt_adj, src2, w0a, w1a, w0b, w1b, rows_a, rows_b,
                  acc_sh, sga, sgb, swa0, swa1, swb0, swb1, ssa, ssb):
    cid = lax.axis_index("c")
    sid = lax.axis_index("s")
    wid = sid * NC + cid
    wbase = pl.multiple_of(wid * EPW, 8)

    pltpu.sync_copy(tgt_hbm.at[pl.ds(wbase, EPW)], tgt_adj)
    pltpu.sync_copy(src2d_hbm.at[wid], src2)

    stripe = NP // NS
    sbase = sid * stripe

    def _issue(c_dyn, p, rows, w0, w1, sg, sw0, sw1):
        coff = pl.multiple_of(c_dyn * SUB2, 8)
        g = pltpu.async_copy(h2_hbm.at[tgt_adj.at[pl.ds(coff, SUB2)]],
                             rows, sg)
        a0 = pltpu.async_copy(
            w_hbm.at[pl.ds((2 * p) * E + wbase + coff, SUB2)], w0, sw0)
        a1 = pltpu.async_copy(
            w_hbm.at[pl.ds((2 * p + 1) * E + wbase + coff, SUB2)], w1, sw1)
        return g, a0, a1

    def _scale(rbuf, w0_v, w1_v):
        for g in range(SUB2 // 16):
            w0g = w0_v[pl.ds(16 * g, 16)]
            w1g = w1_v[pl.ds(16 * g, 16)]
            for l in range(16):
                ei = 16 * g + l
                ws0 = w0g[l]
                ws1 = w1g[l]
                for j in range(FP // 16):
                    ws = ws0 if j < F // 16 else ws1
                    rbuf[ei, pl.ds(16 * j, 16)] = (
                        rbuf[ei, pl.ds(16 * j, 16)] * ws)

    def _wait(rows, w0, w1, sg, sw0, sw1):
        pltpu.make_async_copy(
            h2_hbm.at[tgt_adj.at[pl.ds(0, SUB2)]], rows, sg).wait()
        pltpu.make_async_copy(w_hbm.at[pl.ds(0, SUB2)], w0, sw0).wait()
        pltpu.make_async_copy(w_hbm.at[pl.ds(0, SUB2)], w1, sw1).wait()

    def _wait_scat(rows, ss):
        pltpu.make_async_copy(rows, acc_sh.at[src2.at[0]], ss).wait()

    for p in range(P):
        if p > 0:
            @plsc.parallel_loop(0, EPW // 16, 1, unroll=4)
            def _adj(i):
                tgt_adj[pl.ds(16 * i, 16)] = tgt_adj[pl.ds(16 * i, 16)] + NP

        @plsc.parallel_loop(0, SUB2, 1, unroll=2)
        def _zero_rv(i):
            for j in range(FP // 16):
                rows_b[i, pl.ds(16 * j, 16)] = jnp.zeros((16,), jnp.float32)
        for zo in range(0, stripe, SUB2):
            pltpu.sync_copy(rows_b, acc_sh.at[pl.ds(sbase + zo, SUB2)])
        plsc.subcore_barrier()

        _issue(0, p, rows_a, w0a, w1a, sga, swa0, swa1)

        def _pair(i, carry):
            c0 = 2 * i
            _issue(c0 + 1, p, rows_b, w0b, w1b, sgb, swb0, swb1)
            _wait(rows_a, w0a, w1a, sga, swa0, swa1)
            _scale(rows_a, w0a, w1a)
            pltpu.sync_copy(rows_a, acc_sh.at[src2.at[c0]], add=True)
            _issue(c0 + 2, p, rows_a, w0a, w1a, sga, swa0, swa1)
            _wait(rows_b, w0b, w1b, sgb, swb0, swb1)
            _scale(rows_b, w0b, w1b)
            pltpu.sync_copy(rows_b, acc_sh.at[src2.at[c0 + 1]], add=True)
            return carry
        lax.fori_loop(0, NSUB2 // 2, _pair, 0)
        _wait(rows_a, w0a, w1a, sga, swa0, swa1)
        _scale(rows_a, w0a, w1a)
        pltpu.sync_copy(rows_a, acc_sh.at[src2.at[NSUB2 - 1]], add=True)

        plsc.subcore_barrier()

        pltpu.sync_copy(acc_sh.at[pl.ds(sbase, stripe)],
                        acc_out.at[pl.ds((p * NC + cid) * NP + sbase, stripe)])
        plsc.subcore_barrier()


def _sc_rows(P):
    mesh = plsc.VectorSubcoreMesh(core_axis_name="c", subcore_axis_name="s")
    return pl.kernel(
        functools.partial(_sc_rows_body, P),
        mesh=mesh,
        compiler_params=pltpu.CompilerParams(needs_layout_passes=False),
        out_type=jax.ShapeDtypeStruct((P * NC * NP, FP), jnp.float32),
        scratch_types=[
            pltpu.VMEM((EPW,), jnp.int32),
            pltpu.VMEM((NSUB2, SUB2), jnp.int32),
            pltpu.VMEM((SUB2,), jnp.float32),
            pltpu.VMEM((SUB2,), jnp.float32),
            pltpu.VMEM((SUB2,), jnp.float32),
            pltpu.VMEM((SUB2,), jnp.float32),
            pltpu.VMEM((SUB2, FP), jnp.float32),
            pltpu.VMEM((SUB2, FP), jnp.float32),
            pltpu.VMEM_SHARED((NP, FP), jnp.float32),
            pltpu.SemaphoreType.DMA,
            pltpu.SemaphoreType.DMA,
            pltpu.SemaphoreType.DMA,
            pltpu.SemaphoreType.DMA,
            pltpu.SemaphoreType.DMA,
            pltpu.SemaphoreType.DMA,
            pltpu.SemaphoreType.DMA,
            pltpu.SemaphoreType.DMA,
        ],
    )


def _tc_epi_fn(H, act, acc_r, es_r, b_r, out_r):
    for h in range(H):
        p, half = divmod(h, 2)
        num = (acc_r[2 * p, :, F * half:F * (half + 1)]
               + acc_r[2 * p + 1, :, F * half:F * (half + 1)])
        den = jnp.sum(es_r[pl.ds(NW * h, NW)], axis=0)
        den = den[:, None]
        hp = jnp.where(den > 0, num / den, 0.0) + b_r[h]
        if act == "elu":
            hp = jnp.where(hp > 0, hp, jnp.exp(hp) - 1.0)
            out_r[:, F * h:F * (h + 1)] = hp
        else:
            out_r[...] = jax.nn.sigmoid(hp)


def _tc_epi(H, act, BN=2048):
    P = (H + 1) // 2
    NB = NP // BN
    return pl.pallas_call(
        functools.partial(_tc_epi_fn, H, act),
        grid=(NB,),
        in_specs=[
            pl.BlockSpec((P * NC, BN, FP), lambda i: (0, i, 0)),
            pl.BlockSpec((2 * P * NW, BN), lambda i: (0, i)),
            pl.BlockSpec((H, F), lambda i: (0, 0)),
        ],
        out_specs=pl.BlockSpec((BN, F * (H if act == "elu" else 1)),
                               lambda i: (i, 0)),
        out_shape=jax.ShapeDtypeStruct((NP, F * (H if act == "elu" else 1)),
                                       jnp.float32),
    )


def _tc_fused_fn(H1, H2, P2, NB, acc_r, es_r, b_r, w_r, asrc_r, atgt_r,
                 h_r, ss_r, st_r, m_r, mx_s):
    i = pl.program_id(0)
    cols = []
    for h in range(H1):
        p, half = divmod(h, 2)
        num = (acc_r[2 * p, :, F * half:F * (half + 1)]
               + acc_r[2 * p + 1, :, F * half:F * (half + 1)])
        den = jnp.sum(es_r[pl.ds(NW * h, NW)], axis=0)[:, None]
        hp = jnp.where(den > 0, num / den, 0.0) + b_r[h]
        cols.append(jnp.where(hp > 0, hp, jnp.exp(hp) - 1.0))
    x = jnp.concatenate(cols, axis=1)

    for hh in range(2 * P2):
        p, half = divmod(hh, 2)
        if hh < H2:
            hb = jnp.dot(x, w_r[hh], preferred_element_type=jnp.float32)
            h_r[p, :, F * half:F * (half + 1)] = hb
            s1 = jnp.dot(hb, asrc_r[hh], preferred_element_type=jnp.float32)
            s2 = jnp.dot(hb, atgt_r[hh], preferred_element_type=jnp.float32)
            b1 = jnp.max(s1)
            b2 = jnp.max(s2)
        else:
            h_r[p, :, F * half:F * (half + 1)] = jnp.zeros(
                (x.shape[0], F), jnp.float32)
            s1 = jnp.zeros((x.shape[0],), jnp.float32)
            s2 = s1
            b1 = jnp.float32(0.0)
            b2 = jnp.float32(0.0)
        ss_r[hh] = s1
        st_r[hh] = s2

        @pl.when(i == 0)
        def _():
            mx_s[hh, 0] = b1
            mx_s[hh, 1] = b2

        @pl.when(i > 0)
        def _():
            mx_s[hh, 0] = jnp.maximum(mx_s[hh, 0], b1)
            mx_s[hh, 1] = jnp.maximum(mx_s[hh, 1], b2)

    @pl.when(i == NB - 1)
    def _():
        for hh in range(2 * P2):
            m_r[hh] = jnp.full((128,),
                               jnp.maximum(mx_s[hh, 0] + mx_s[hh, 1], 0.0),
                               dtype=jnp.float32)


def _tc_fused(H1, H2, K2, BN=2048):
    P1 = (H1 + 1) // 2
    P2 = (H2 + 1) // 2
    NB = NP // BN
    return pl.pallas_call(
        functools.partial(_tc_fused_fn, H1, H2, P2, NB),
        grid=(NB,),
        in_specs=[
            pl.BlockSpec((P1 * NC, BN, FP), lambda i: (0, i, 0)),
            pl.BlockSpec((2 * P1 * NW, BN), lambda i: (0, i)),
            pl.BlockSpec((H1, F), lambda i: (0, 0)),
            pl.BlockSpec((H2, K2, F), lambda i: (0, 0, 0)),
            pl.BlockSpec((H2, F), lambda i: (0, 0)),
            pl.BlockSpec((H2, F), lambda i: (0, 0)),
        ],
        out_specs=[
            pl.BlockSpec((P2, BN, FP), lambda i: (0, i, 0)),
            pl.BlockSpec((2 * P2, BN), lambda i: (0, i)),
            pl.BlockSpec((2 * P2, BN), lambda i: (0, i)),
            pl.BlockSpec((2 * P2, 128), lambda i: (0, 0)),
        ],
        out_shape=[
            jax.ShapeDtypeStruct((P2, NP, FP), jnp.float32),
            jax.ShapeDtypeStruct((2 * P2, NP), jnp.float32),
            jax.ShapeDtypeStruct((2 * P2, NP), jnp.float32),
            jax.ShapeDtypeStruct((2 * P2, 128), jnp.float32),
        ],
        scratch_shapes=[pltpu.SMEM((2 * P2, 2), jnp.float32)],
    )


def _sc_stage(P, h3, ssrc, stgt, m, src, src2d, tgt):
    h2 = h3.reshape(P * NP, FP)
    w_e, es = _sc_scores(P)(ssrc, stgt, m, src, tgt)
    acc_flat = _sc_rows(P)(h2, src2d, tgt, w_e)
    return acc_flat.reshape(P * NC, NP, FP), es


def _split_a(a):
    return a[:, :F, 0], a[:, F:, 0]


def kernel(x, edge_index, W1, a1, b1, W2, a2, b2, W3, a3, b3):
    src = edge_index[0].astype(jnp.int32)
    tgt = edge_index[1].astype(jnp.int32)
    src2d = src.reshape(NW, NSUB2, SUB2)
    xp = jnp.pad(x, ((0, NP - N), (0, 0)))

    a1s, a1t = _split_a(a1)
    a2s, a2t = _split_a(a2)
    a3s, a3t = _split_a(a3)

    h3, ss, st, m = _tc_head(4, 128)(xp, W1, a1s, a1t)
    acc1, es1 = _sc_stage(2, h3, ss, st, m, src, src2d, tgt)

    h3, ss, st, m = _tc_fused(4, 4, 256)(acc1, es1, b1, W2, a2s, a2t)
    acc2, es2 = _sc_stage(2, h3, ss, st, m, src, src2d, tgt)

    h3, ss, st, m = _tc_fused(4, 1, 256)(acc2, es2, b2, W3, a3s, a3t)
    acc3, es3 = _sc_stage(1, h3, ss, st, m, src, src2d, tgt)

    out = _tc_epi(1, "sigmoid")(acc3, es3, b3)
    return out[:N]


# speedup vs baseline: 1.0204x; 1.0002x over previous
import functools

import jax
import jax.numpy as jnp
from jax import lax
from jax.experimental import pallas as pl
from jax.experimental.pallas import tpu as pltpu
from jax.experimental.pallas import tpu_sc as plsc

N = 10000
NP = 10240
E = 320000
F = 64
FP = 2 * F
ALPHA = 0.2

NC = 2
NS = 16
NW = NC * NS
EPW = E // NW
SUB2 = 80
NSUB2 = EPW // SUB2


def _tc_head_fn(H, P, NB, x_r, w_r, asrc_r, atgt_r, h_r, ss_r, st_r, m_r,
                mx_s):
    i = pl.program_id(0)
    for hh in range(2 * P):
        p, half = divmod(hh, 2)
        if hh < H:
            hb = jnp.dot(x_r[...], w_r[hh], preferred_element_type=jnp.float32)
            h_r[p, :, F * half:F * (half + 1)] = hb
            s1 = jnp.dot(hb, asrc_r[hh], preferred_element_type=jnp.float32)
            s2 = jnp.dot(hb, atgt_r[hh], preferred_element_type=jnp.float32)
            b1 = jnp.max(s1)
            b2 = jnp.max(s2)
        else:
            h_r[p, :, F * half:F * (half + 1)] = jnp.zeros(
                (x_r.shape[0], F), jnp.float32)
            s1 = jnp.zeros((x_r.shape[0],), jnp.float32)
            s2 = s1
            b1 = jnp.float32(0.0)
            b2 = jnp.float32(0.0)
        ss_r[hh] = s1
        st_r[hh] = s2

        @pl.when(i == 0)
        def _():
            mx_s[hh, 0] = b1
            mx_s[hh, 1] = b2

        @pl.when(i > 0)
        def _():
            mx_s[hh, 0] = jnp.maximum(mx_s[hh, 0], b1)
            mx_s[hh, 1] = jnp.maximum(mx_s[hh, 1], b2)

    @pl.when(i == NB - 1)
    def _():
        for hh in range(2 * P):
            m_r[hh] = jnp.full((128,),
                               jnp.maximum(mx_s[hh, 0] + mx_s[hh, 1], 0.0),
                               dtype=jnp.float32)


def _tc_head(H, K, BN=2048):
    P = (H + 1) // 2
    NB = NP // BN
    return pl.pallas_call(
        functools.partial(_tc_head_fn, H, P, NB),
        grid=(NB,),
        in_specs=[
            pl.BlockSpec((BN, K), lambda i: (i, 0)),
            pl.BlockSpec((H, K, F), lambda i: (0, 0, 0)),
            pl.BlockSpec((H, F), lambda i: (0, 0)),
            pl.BlockSpec((H, F), lambda i: (0, 0)),
        ],
        out_specs=[
            pl.BlockSpec((P, BN, FP), lambda i: (0, i, 0)),
            pl.BlockSpec((2 * P, BN), lambda i: (0, i)),
            pl.BlockSpec((2 * P, BN), lambda i: (0, i)),
            pl.BlockSpec((2 * P, 128), lambda i: (0, 0)),
        ],
        out_shape=[
            jax.ShapeDtypeStruct((P, NP, FP), jnp.float32),
            jax.ShapeDtypeStruct((2 * P, NP), jnp.float32),
            jax.ShapeDtypeStruct((2 * P, NP), jnp.float32),
            jax.ShapeDtypeStruct((2 * P, 128), jnp.float32),
        ],
        scratch_shapes=[pltpu.SMEM((2 * P, 2), jnp.float32)],
    )


def _sc_scores_body(P, ssrc_hbm, stgt_hbm, m_hbm, src_hbm, tgt_hbm,
                    w_out, es_out,
                    ssrc0_t, stgt0_t, ssrc1_t, stgt1_t, es0_t, es1_t,
                    src_w, tgt_w, w0_b, w1_b, m_v):
    cid = lax.axis_index("c")
    sid = lax.axis_index("s")
    wid = sid * NC + cid
    wbase = pl.multiple_of(wid * EPW, 8)

    pltpu.sync_copy(src_hbm.at[pl.ds(wbase, EPW)], src_w)
    pltpu.sync_copy(tgt_hbm.at[pl.ds(wbase, EPW)], tgt_w)

    for p in range(P):
        pltpu.sync_copy(ssrc_hbm.at[2 * p], ssrc0_t)
        pltpu.sync_copy(stgt_hbm.at[2 * p], stgt0_t)
        pltpu.sync_copy(ssrc_hbm.at[2 * p + 1], ssrc1_t)
        pltpu.sync_copy(stgt_hbm.at[2 * p + 1], stgt1_t)
        pltpu.sync_copy(m_hbm.at[2 * p], m_v)
        m0 = m_v[pl.ds(0, 16)]
        pltpu.sync_copy(m_hbm.at[2 * p + 1], m_v)
        m1 = m_v[pl.ds(0, 16)]

        @plsc.parallel_loop(0, NP // 16, 1, unroll=4)
        def _zero_es(i):
            es0_t[pl.ds(16 * i, 16)] = jnp.zeros((16,), jnp.float32)
            es1_t[pl.ds(16 * i, 16)] = jnp.zeros((16,), jnp.float32)

        @plsc.parallel_loop(0, EPW // 16, 1, unroll=2)
        def _group(g):
            goff = pl.multiple_of(g * 16, 8)
            i_s = src_w[pl.ds(goff, 16)]
            i_t = tgt_w[pl.ds(goff, 16)]
            s1 = plsc.load_gather(ssrc0_t, [i_s])
            s2 = plsc.load_gather(stgt0_t, [i_t])
            e0 = s1 + s2
            e0 = jnp.where(e0 > 0, e0, ALPHA * e0)
            w0 = jnp.exp(e0 - m0)
            w0_b[pl.ds(goff, 16)] = w0
            plsc.addupdate_scatter(es0_t, [i_s], w0)
            s3 = plsc.load_gather(ssrc1_t, [i_s])
            s4 = plsc.load_gather(stgt1_t, [i_t])
            e1 = s3 + s4
            e1 = jnp.where(e1 > 0, e1, ALPHA * e1)
            w1 = jnp.exp(e1 - m1)
            w1_b[pl.ds(goff, 16)] = w1
            plsc.addupdate_scatter(es1_t, [i_s], w1)

        pltpu.sync_copy(w0_b, w_out.at[pl.ds((2 * p) * E + wbase, EPW)])
        pltpu.sync_copy(w1_b, w_out.at[pl.ds((2 * p + 1) * E + wbase, EPW)])
        pltpu.sync_copy(es0_t, es_out.at[(2 * p) * NW + wid])
        pltpu.sync_copy(es1_t, es_out.at[(2 * p + 1) * NW + wid])


def _sc_scores(P):
    mesh = plsc.VectorSubcoreMesh(core_axis_name="c", subcore_axis_name="s")
    return pl.kernel(
        functools.partial(_sc_scores_body, P),
        mesh=mesh,
        compiler_params=pltpu.CompilerParams(needs_layout_passes=False),
        out_type=[
            jax.ShapeDtypeStruct((2 * P * E,), jnp.float32),
            jax.ShapeDtypeStruct((2 * P * NW, NP), jnp.float32),
        ],
        scratch_types=[
            pltpu.VMEM((NP,), jnp.float32),
            pltpu.VMEM((NP,), jnp.float32),
            pltpu.VMEM((NP,), jnp.float32),
            pltpu.VMEM((NP,), jnp.float32),
            pltpu.VMEM((NP,), jnp.float32),
            pltpu.VMEM((NP,), jnp.float32),
            pltpu.VMEM((EPW,), jnp.int32),
            pltpu.VMEM((EPW,), jnp.int32),
            pltpu.VMEM((EPW,), jnp.float32),
            pltpu.VMEM((EPW,), jnp.float32),
            pltpu.VMEM((128,), jnp.float32),
        ],
    )


def _sc_rows_body(P, h2_hbm, src2d_hbm, tgt_hbm, w_hbm, acc_out,
                  tgt_adj, src2, w0a, w1a, w0b, w1b, rows_a, rows_b,
                  acc_sh, sga, sgb, swa0, swa1, swb0, swb1):
    cid = lax.axis_index("c")
    sid = lax.axis_index("s")
    wid = sid * NC + cid
    wbase = pl.multiple_of(wid * EPW, 8)

    pltpu.sync_copy(tgt_hbm.at[pl.ds(wbase, EPW)], tgt_adj)
    pltpu.sync_copy(src2d_hbm.at[wid], src2)

    stripe = NP // NS
    sbase = sid * stripe

    def _issue(c_dyn, p, rows, w0, w1, sg, sw0, sw1):
        coff = pl.multiple_of(c_dyn * SUB2, 8)
        g = pltpu.async_copy(h2_hbm.at[tgt_adj.at[pl.ds(coff, SUB2)]],
                             rows, sg)
        a0 = pltpu.async_copy(
            w_hbm.at[pl.ds((2 * p) * E + wbase + coff, SUB2)], w0, sw0)
        a1 = pltpu.async_copy(
            w_hbm.at[pl.ds((2 * p + 1) * E + wbase + coff, SUB2)], w1, sw1)
        return g, a0, a1

    def _scale(rbuf, w0_v, w1_v):
        for g in range(SUB2 // 16):
            w0g = w0_v[pl.ds(16 * g, 16)]
            w1g = w1_v[pl.ds(16 * g, 16)]
            for l in range(16):
                ei = 16 * g + l
                ws0 = w0g[l]
                ws1 = w1g[l]
                for j in range(FP // 16):
                    ws = ws0 if j < F // 16 else ws1
                    rbuf[ei, pl.ds(16 * j, 16)] = (
                        rbuf[ei, pl.ds(16 * j, 16)] * ws)

    def _wait(rows, w0, w1, sg, sw0, sw1):
        pltpu.make_async_copy(
            h2_hbm.at[tgt_adj.at[pl.ds(0, SUB2)]], rows, sg).wait()
        pltpu.make_async_copy(w_hbm.at[pl.ds(0, SUB2)], w0, sw0).wait()
        pltpu.make_async_copy(w_hbm.at[pl.ds(0, SUB2)], w1, sw1).wait()

    for p in range(P):
        if p > 0:
            @plsc.parallel_loop(0, EPW // 16, 1, unroll=4)
            def _adj(i):
                tgt_adj[pl.ds(16 * i, 16)] = tgt_adj[pl.ds(16 * i, 16)] + NP

        @plsc.parallel_loop(0, SUB2, 1, unroll=2)
        def _zero_rv(i):
            for j in range(FP // 16):
                rows_b[i, pl.ds(16 * j, 16)] = jnp.zeros((16,), jnp.float32)
        for zo in range(0, stripe, SUB2):
            pltpu.sync_copy(rows_b, acc_sh.at[pl.ds(sbase + zo, SUB2)])
        plsc.subcore_barrier()

        _issue(0, p, rows_a, w0a, w1a, sga, swa0, swa1)

        def _pair(i, carry):
            c0 = 2 * i
            _issue(c0 + 1, p, rows_b, w0b, w1b, sgb, swb0, swb1)
            _wait(rows_a, w0a, w1a, sga, swa0, swa1)
            _scale(rows_a, w0a, w1a)
            pltpu.sync_copy(rows_a, acc_sh.at[src2.at[c0]], add=True)
            _issue(c0 + 2, p, rows_a, w0a, w1a, sga, swa0, swa1)
            _wait(rows_b, w0b, w1b, sgb, swb0, swb1)
            _scale(rows_b, w0b, w1b)
            pltpu.sync_copy(rows_b, acc_sh.at[src2.at[c0 + 1]], add=True)
            return carry
        lax.fori_loop(0, NSUB2 // 2, _pair, 0)
        _wait(rows_a, w0a, w1a, sga, swa0, swa1)
        _scale(rows_a, w0a, w1a)
        pltpu.sync_copy(rows_a, acc_sh.at[src2.at[NSUB2 - 1]], add=True)

        plsc.subcore_barrier()

        pltpu.sync_copy(acc_sh.at[pl.ds(sbase, stripe)],
                        acc_out.at[pl.ds((p * NC + cid) * NP + sbase, stripe)])
        plsc.subcore_barrier()


def _sc_rows(P):
    mesh = plsc.VectorSubcoreMesh(core_axis_name="c", subcore_axis_name="s")
    return pl.kernel(
        functools.partial(_sc_rows_body, P),
        mesh=mesh,
        compiler_params=pltpu.CompilerParams(needs_layout_passes=False),
        out_type=jax.ShapeDtypeStruct((P * NC * NP, FP), jnp.float32),
        scratch_types=[
            pltpu.VMEM((EPW,), jnp.int32),
            pltpu.VMEM((NSUB2, SUB2), jnp.int32),
            pltpu.VMEM((SUB2,), jnp.float32),
            pltpu.VMEM((SUB2,), jnp.float32),
            pltpu.VMEM((SUB2,), jnp.float32),
            pltpu.VMEM((SUB2,), jnp.float32),
            pltpu.VMEM((SUB2, FP), jnp.float32),
            pltpu.VMEM((SUB2, FP), jnp.float32),
            pltpu.VMEM_SHARED((NP, FP), jnp.float32),
            pltpu.SemaphoreType.DMA,
            pltpu.SemaphoreType.DMA,
            pltpu.SemaphoreType.DMA,
            pltpu.SemaphoreType.DMA,
            pltpu.SemaphoreType.DMA,
            pltpu.SemaphoreType.DMA,
        ],
    )


def _tc_epi_fn(H, act, acc_r, es_r, b_r, out_r):
    for h in range(H):
        p, half = divmod(h, 2)
        num = (acc_r[2 * p, :, F * half:F * (half + 1)]
               + acc_r[2 * p + 1, :, F * half:F * (half + 1)])
        den = jnp.sum(es_r[pl.ds(NW * h, NW)], axis=0)
        den = den[:, None]
        hp = jnp.where(den > 0, num / den, 0.0) + b_r[h]
        if act == "elu":
            hp = jnp.where(hp > 0, hp, jnp.exp(hp) - 1.0)
            out_r[:, F * h:F * (h + 1)] = hp
        else:
            out_r[...] = jax.nn.sigmoid(hp)


def _tc_epi(H, act, BN=2048):
    P = (H + 1) // 2
    NB = NP // BN
    return pl.pallas_call(
        functools.partial(_tc_epi_fn, H, act),
        grid=(NB,),
        in_specs=[
            pl.BlockSpec((P * NC, BN, FP), lambda i: (0, i, 0)),
            pl.BlockSpec((2 * P * NW, BN), lambda i: (0, i)),
            pl.BlockSpec((H, F), lambda i: (0, 0)),
        ],
        out_specs=pl.BlockSpec((BN, F * (H if act == "elu" else 1)),
                               lambda i: (i, 0)),
        out_shape=jax.ShapeDtypeStruct((NP, F * (H if act == "elu" else 1)),
                                       jnp.float32),
    )


def _tc_fused_fn(H1, H2, P2, NB, acc_r, es_r, b_r, w_r, asrc_r, atgt_r,
                 h_r, ss_r, st_r, m_r, mx_s):
    i = pl.program_id(0)
    cols = []
    for h in range(H1):
        p, half = divmod(h, 2)
        num = (acc_r[2 * p, :, F * half:F * (half + 1)]
               + acc_r[2 * p + 1, :, F * half:F * (half + 1)])
        den = jnp.sum(es_r[pl.ds(NW * h, NW)], axis=0)[:, None]
        hp = jnp.where(den > 0, num / den, 0.0) + b_r[h]
        cols.append(jnp.where(hp > 0, hp, jnp.exp(hp) - 1.0))
    x = jnp.concatenate(cols, axis=1)

    for hh in range(2 * P2):
        p, half = divmod(hh, 2)
        if hh < H2:
            hb = jnp.dot(x, w_r[hh], preferred_element_type=jnp.float32)
            h_r[p, :, F * half:F * (half + 1)] = hb
            s1 = jnp.dot(hb, asrc_r[hh], preferred_element_type=jnp.float32)
            s2 = jnp.dot(hb, atgt_r[hh], preferred_element_type=jnp.float32)
            b1 = jnp.max(s1)
            b2 = jnp.max(s2)
        else:
            h_r[p, :, F * half:F * (half + 1)] = jnp.zeros(
                (x.shape[0], F), jnp.float32)
            s1 = jnp.zeros((x.shape[0],), jnp.float32)
            s2 = s1
            b1 = jnp.float32(0.0)
            b2 = jnp.float32(0.0)
        ss_r[hh] = s1
        st_r[hh] = s2

        @pl.when(i == 0)
        def _():
            mx_s[hh, 0] = b1
            mx_s[hh, 1] = b2

        @pl.when(i > 0)
        def _():
            mx_s[hh, 0] = jnp.maximum(mx_s[hh, 0], b1)
            mx_s[hh, 1] = jnp.maximum(mx_s[hh, 1], b2)

    @pl.when(i == NB - 1)
    def _():
        for hh in range(2 * P2):
            m_r[hh] = jnp.full((128,),
                               jnp.maximum(mx_s[hh, 0] + mx_s[hh, 1], 0.0),
                               dtype=jnp.float32)


def _tc_fused(H1, H2, K2, BN=2048):
    P1 = (H1 + 1) // 2
    P2 = (H2 + 1) // 2
    NB = NP // BN
    return pl.pallas_call(
        functools.partial(_tc_fused_fn, H1, H2, P2, NB),
        grid=(NB,),
        in_specs=[
            pl.BlockSpec((P1 * NC, BN, FP), lambda i: (0, i, 0)),
            pl.BlockSpec((2 * P1 * NW, BN), lambda i: (0, i)),
            pl.BlockSpec((H1, F), lambda i: (0, 0)),
            pl.BlockSpec((H2, K2, F), lambda i: (0, 0, 0)),
            pl.BlockSpec((H2, F), lambda i: (0, 0)),
            pl.BlockSpec((H2, F), lambda i: (0, 0)),
        ],
        out_specs=[
            pl.BlockSpec((P2, BN, FP), lambda i: (0, i, 0)),
            pl.BlockSpec((2 * P2, BN), lambda i: (0, i)),
            pl.BlockSpec((2 * P2, BN), lambda i: (0, i)),
            pl.BlockSpec((2 * P2, 128), lambda i: (0, 0)),
        ],
        out_shape=[
            jax.ShapeDtypeStruct((P2, NP, FP), jnp.float32),
            jax.ShapeDtypeStruct((2 * P2, NP), jnp.float32),
            jax.ShapeDtypeStruct((2 * P2, NP), jnp.float32),
            jax.ShapeDtypeStruct((2 * P2, 128), jnp.float32),
        ],
        scratch_shapes=[pltpu.SMEM((2 * P2, 2), jnp.float32)],
    )


def _sc_stage(P, h3, ssrc, stgt, m, src, src2d, tgt):
    h2 = h3.reshape(P * NP, FP)
    w_e, es = _sc_scores(P)(ssrc, stgt, m, src, tgt)
    acc_flat = _sc_rows(P)(h2, src2d, tgt, w_e)
    return acc_flat.reshape(P * NC, NP, FP), es


def _split_a(a):
    return a[:, :F, 0], a[:, F:, 0]


def kernel(x, edge_index, W1, a1, b1, W2, a2, b2, W3, a3, b3):
    src = edge_index[0].astype(jnp.int32)
    tgt = edge_index[1].astype(jnp.int32)
    src2d = src.reshape(NW, NSUB2, SUB2)
    xp = jnp.pad(x, ((0, NP - N), (0, 0)))

    a1s, a1t = _split_a(a1)
    a2s, a2t = _split_a(a2)
    a3s, a3t = _split_a(a3)

    h3, ss, st, m = _tc_head(4, 128)(xp, W1, a1s, a1t)
    acc1, es1 = _sc_stage(2, h3, ss, st, m, src, src2d, tgt)

    h3, ss, st, m = _tc_fused(4, 4, 256)(acc1, es1, b1, W2, a2s, a2t)
    acc2, es2 = _sc_stage(2, h3, ss, st, m, src, src2d, tgt)

    h3, ss, st, m = _tc_fused(4, 1, 256)(acc2, es2, b2, W3, a3s, a3t)
    acc3, es3 = _sc_stage(1, h3, ss, st, m, src, src2d, tgt)

    out = _tc_epi(1, "sigmoid")(acc3, es3, b3)
    return out[:N]
